# Initial kernel scaffold; baseline (speedup 1.0000x reference)
import jax
import jax.numpy as jnp
from jax.experimental import pallas as pl


def kernel(x, edge_index, x_scalar, batch_index, Wl1, Wr1, att1, b1, Wl2, Wr2, att2, b2):
    raise NotImplementedError("write your pallas kernel here")


# speedup vs baseline: 48.9601x; 48.9601x over previous
import functools

import jax
import jax.numpy as jnp
from jax import lax
from jax.experimental import pallas as pl
from jax.experimental.pallas import tpu as pltpu
from jax.experimental.pallas import tpu_sc as plsc

N = 10000
E = 320000
G = 64
HID = 64
ROWW = 128
CHUNK = 128
NCHUNKS = E // CHUNK
NCORES = 2
NSUB = 16
NW = NCORES * NSUB
ROW_STRIDE = 624

_mesh = plsc.VectorSubcoreMesh(core_axis_name="c", subcore_axis_name="s")


def _hsum(v):
    lane = lax.iota(jnp.int32, 16)
    dnums = lax.GatherDimensionNumbers(
        offset_dims=(), collapsed_slice_dims=(0,), start_index_map=(0,))
    for sh in (8, 4, 2, 1):
        idx = (lane + sh) % 16
        rot = lax.gather(v, idx[:, None], dimension_numbers=dnums,
                         slice_sizes=(1,),
                         mode=lax.GatherScatterMode.PROMISE_IN_BOUNDS)
        v = v + rot
    return v


@functools.partial(
    pl.kernel,
    mesh=_mesh,
    out_type=jax.ShapeDtypeStruct((NCORES, N, ROWW), jnp.float32),
    scratch_types=[
        pltpu.VMEM((CHUNK,), jnp.int32),
        pltpu.VMEM((CHUNK,), jnp.int32),
        pltpu.VMEM((CHUNK, 2 * HID), jnp.float32),
        pltpu.VMEM((CHUNK, 2 * HID), jnp.float32),
        pltpu.VMEM((CHUNK, ROWW), jnp.float32),
        pltpu.VMEM((HID,), jnp.float32),
        pltpu.VMEM_SHARED((N, ROWW), jnp.float32),
        pltpu.SemaphoreType.DMA,
        pltpu.SemaphoreType.DMA,
    ],
)
def _edge_pass(xcat_hbm, src_hbm, dst_hbm, att_hbm, out_hbm,
               idx_s, idx_d, xl_rows, xr_rows, scaled, att_v, accum,
               sem1, sem2):
    c = lax.axis_index("c")
    s = lax.axis_index("s")
    wid = s * NCORES + c

    def _zrow(r, carry):
        for k in range(ROWW // 16):
            scaled[r, pl.ds(k * 16, 16)] = jnp.zeros((16,), jnp.float32)
        return carry
    lax.fori_loop(0, CHUNK, _zrow, 0)
    for k in range(5):
        pltpu.sync_copy(scaled.at[pl.ds(0, CHUNK)],
                        accum.at[pl.ds(s * ROW_STRIDE + k * CHUNK, CHUNK)])
    pltpu.sync_copy(att_hbm, att_v)
    plsc.subcore_barrier()

    n_j = jnp.where(wid < NCHUNKS % NW, NCHUNKS // NW + 1, NCHUNKS // NW)

    def _chunk(j, carry):
        base = (wid + j * NW) * CHUNK
        pltpu.sync_copy(src_hbm.at[pl.ds(base, CHUNK)], idx_s)
        pltpu.sync_copy(dst_hbm.at[pl.ds(base, CHUNK)], idx_d)
        cp1 = pltpu.async_copy(xcat_hbm.at[idx_s], xl_rows, sem1)
        cp2 = pltpu.async_copy(xcat_hbm.at[idx_d], xr_rows, sem2)
        cp1.wait()
        cp2.wait()

        def _edge(e, ecarry):
            a0 = att_v[pl.ds(0, 16)]
            a1 = att_v[pl.ds(16, 16)]
            a2 = att_v[pl.ds(32, 16)]
            a3 = att_v[pl.ds(48, 16)]
            l0 = xl_rows[e, pl.ds(0, 16)]
            l1 = xl_rows[e, pl.ds(16, 16)]
            l2 = xl_rows[e, pl.ds(32, 16)]
            l3 = xl_rows[e, pl.ds(48, 16)]
            r0 = xr_rows[e, pl.ds(64, 16)]
            r1 = xr_rows[e, pl.ds(80, 16)]
            r2 = xr_rows[e, pl.ds(96, 16)]
            r3 = xr_rows[e, pl.ds(112, 16)]
            s0 = l0 + r0
            s1 = l1 + r1
            s2 = l2 + r2
            s3 = l3 + r3
            z0 = jnp.maximum(s0, 0.2 * s0)
            z1 = jnp.maximum(s1, 0.2 * s1)
            z2 = jnp.maximum(s2, 0.2 * s2)
            z3 = jnp.maximum(s3, 0.2 * s3)
            e0 = _hsum(z0 * a0 + z1 * a1)
            e1 = _hsum(z2 * a2 + z3 * a3)
            ex0 = jnp.exp(e0)
            ex1 = jnp.exp(e1)
            scaled[e, pl.ds(0, 16)] = l0 * ex0
            scaled[e, pl.ds(16, 16)] = l1 * ex0
            scaled[e, pl.ds(32, 16)] = l2 * ex1
            scaled[e, pl.ds(48, 16)] = l3 * ex1
            lane = lax.iota(jnp.int32, 16)
            excol = jnp.where(lane == 0, ex0,
                              jnp.where(lane == 1, ex1,
                                        jnp.zeros((16,), jnp.float32)))
            scaled[e, pl.ds(64, 16)] = excol
            return ecarry

        lax.fori_loop(0, CHUNK, _edge, 0)
        pltpu.sync_copy(scaled, accum.at[idx_d], add=True)
        return carry

    lax.fori_loop(0, n_j, _chunk, 0)
    plsc.subcore_barrier()

    for k in range(5):
        pltpu.sync_copy(accum.at[pl.ds(s * ROW_STRIDE + k * CHUNK, CHUNK)],
                        out_hbm.at[c, pl.ds(s * ROW_STRIDE + k * CHUNK, CHUNK)])


def _mm(x, Wl, Wr):
    n, k = x.shape
    blk = 1000

    def body(x_ref, w_ref, out_ref):
        out_ref[...] = jnp.dot(x_ref[...], w_ref[...],
                               preferred_element_type=jnp.float32)

    W = jnp.concatenate([Wl, Wr], axis=1)
    return pl.pallas_call(
        body,
        grid=(n // blk,),
        in_specs=[
            pl.BlockSpec((blk, k), lambda i: (i, 0)),
            pl.BlockSpec((k, 2 * HID), lambda i: (0, 0)),
        ],
        out_specs=pl.BlockSpec((blk, 2 * HID), lambda i: (i, 0)),
        out_shape=jax.ShapeDtypeStruct((n, 2 * HID), jnp.float32),
    )(x, W)


def _selfloop_merge(xcat, part, attrow, brow):
    xl = xcat[:, :HID]
    xr = xcat[:, HID:]
    p0 = part[0]
    p1 = part[1]
    acc = p0[:, :HID] + p1[:, :HID]
    sl = xl + xr
    z = jnp.maximum(sl, 0.2 * sl)
    w = z * attrow
    lane = lax.broadcasted_iota(jnp.int32, (1, HID), 1)
    m0 = (lane < 32).astype(jnp.float32)
    m1 = 1.0 - m0
    e0 = jnp.sum(w * m0, axis=1, keepdims=True)
    e1 = jnp.sum(w * m1, axis=1, keepdims=True)
    ex0 = jnp.exp(e0)
    ex1 = jnp.exp(e1)
    acc = acc + xl * (ex0 * m0 + ex1 * m1)
    d0 = p0[:, 64:65] + p1[:, 64:65] + ex0
    d1 = p0[:, 65:66] + p1[:, 65:66] + ex1
    den = d0 * m0 + d1 * m1
    h = acc / den + brow
    return jnp.maximum(h, 0.01 * h)


def _combine(xcat, part, attrow, brow):
    blk = 1000

    def body(xc_ref, part_ref, att_ref, b_ref, out_ref):
        out_ref[...] = _selfloop_merge(xc_ref[...], part_ref[...],
                                       att_ref[...], b_ref[...])

    return pl.pallas_call(
        body,
        grid=(N // blk,),
        in_specs=[
            pl.BlockSpec((blk, 2 * HID), lambda i: (i, 0)),
            pl.BlockSpec((NCORES, blk, ROWW), lambda i: (0, i, 0)),
            pl.BlockSpec((1, HID), lambda i: (0, 0)),
            pl.BlockSpec((1, HID), lambda i: (0, 0)),
        ],
        out_specs=pl.BlockSpec((blk, HID), lambda i: (i, 0)),
        out_shape=jax.ShapeDtypeStruct((N, HID), jnp.float32),
    )(xcat, part, attrow, brow)


def _combine_pool(xcat, part, attrow, brow, batch):
    blk = 1000

    def body(xc_ref, part_ref, att_ref, b_ref, batch_ref,
             sums_ref, counts_ref):
        h = _selfloop_merge(xc_ref[...], part_ref[...],
                            att_ref[...], b_ref[...])
        bi = batch_ref[...]
        lane = lax.broadcasted_iota(jnp.int32, (1, G), 1)
        oh = (bi == lane).astype(jnp.float32)
        ps = lax.dot_general(oh, h, (((0,), (0,)), ((), ())),
                             preferred_element_type=jnp.float32)
        ones = jnp.ones((blk, 1), jnp.float32)
        pc = lax.dot_general(oh, ones, (((0,), (0,)), ((), ())),
                             preferred_element_type=jnp.float32)

        @pl.when(pl.program_id(0) == 0)
        def _():
            sums_ref[...] = jnp.zeros_like(sums_ref)
            counts_ref[...] = jnp.zeros_like(counts_ref)

        sums_ref[...] += ps
        counts_ref[...] += pc

    return pl.pallas_call(
        body,
        grid=(N // blk,),
        in_specs=[
            pl.BlockSpec((blk, 2 * HID), lambda i: (i, 0)),
            pl.BlockSpec((NCORES, blk, ROWW), lambda i: (0, i, 0)),
            pl.BlockSpec((1, HID), lambda i: (0, 0)),
            pl.BlockSpec((1, HID), lambda i: (0, 0)),
            pl.BlockSpec((blk, 1), lambda i: (i, 0)),
        ],
        out_specs=[
            pl.BlockSpec((G, HID), lambda i: (0, 0)),
            pl.BlockSpec((G, 1), lambda i: (0, 0)),
        ],
        out_shape=[
            jax.ShapeDtypeStruct((G, HID), jnp.float32),
            jax.ShapeDtypeStruct((G, 1), jnp.float32),
        ],
    )(xcat, part, attrow, brow, batch)


def _final(sums, counts, x_scalar):
    def body(sums_ref, counts_ref, xs_ref, out_ref):
        pooled = sums_ref[...] / jnp.maximum(counts_ref[...], 1.0)
        out_ref[:, :HID] = pooled
        out_ref[:, HID:] = xs_ref[...]

    return pl.pallas_call(
        body,
        out_shape=jax.ShapeDtypeStruct((G, HID + x_scalar.shape[1]), jnp.float32),
    )(sums, counts, x_scalar)


def kernel(x, edge_index, x_scalar, batch_index, Wl1, Wr1, att1, b1,
           Wl2, Wr2, att2, b2):
    att1v = att1.reshape(-1)
    att2v = att2.reshape(-1)
    att1row = att1.reshape(1, HID)
    att2row = att2.reshape(1, HID)
    b1row = b1.reshape(1, HID)
    b2row = b2.reshape(1, HID)

    src = edge_index[0]
    dst = edge_index[1]
    xcat1 = _mm(x, Wl1, Wr1)
    part1 = _edge_pass(xcat1, src, dst, att1v)
    h1 = _combine(xcat1, part1, att1row, b1row)
    xcat2 = _mm(h1, Wl2, Wr2)
    part2 = _edge_pass(xcat2, src, dst, att2v)
    sums, counts = _combine_pool(xcat2, part2, att2row, b2row,
                                 batch_index.reshape(N, 1))
    return _final(sums, counts, x_scalar)
